# baseline (speedup 1.0000x reference)
import functools

import jax
import jax.numpy as jnp
import numpy as np
from jax import lax
from jax.experimental import pallas as pl
from jax.experimental.pallas import tpu as pltpu
from jax.experimental.pallas import tpu_sc as plsc

_EMBED_DIM = 128
_BATCH = 16384
_NUM_FIELDS = 26
_FIELD_DIM = 1000

_NC = 2
_NS = 16
_NW = _NC * _NS
_TOTAL = _BATCH * _NUM_FIELDS
_BATCH_W = _BATCH // _NW
_BLK = 128
_BPF = _BATCH_W // _BLK


def _body(x_hbm, w_hbm, out_hbm, xt_v, spA, spB, rows0, rows1, rows2, rows3,
          gsem, wsem, csem):
    c = lax.axis_index("c")
    s = lax.axis_index("s")
    wid = s * _NC + c
    b0 = wid * _BATCH_W
    rows = (rows0, rows1, rows2, rows3)
    sps = (spA, spB)

    pltpu.sync_copy(x_hbm.at[:, pl.ds(b0, _BATCH_W)], xt_v)

    def fld(i):
        return lax.rem(i + c * (_NUM_FIELDS // 2), _NUM_FIELDS)

    def idx_at(f, j):
        return xt_v.at[f, pl.ds(j * _BLK, _BLK)]

    def gather(sp, f, j, buf):
        pltpu.async_copy(sp.at[idx_at(f, j)], buf, gsem)

    def out_at(f, j):
        return out_hbm.at[pl.ds(f * _BATCH + b0 + j * _BLK, _BLK)]

    def prefetch(i, sp):
        pltpu.async_copy(
            w_hbm.at[pl.ds(fld(i) * _FIELD_DIM, _FIELD_DIM)], sp, csem)

    @pl.when(s == 0)
    def _():
        prefetch(0, sps[0])

    def field_body(i, parity, k):
        sp = sps[parity]
        f = fld(i)

        @pl.when(s == 0)
        def _():
            pltpu.make_async_copy(
                w_hbm.at[pl.ds(0, _FIELD_DIM)], sp, csem).wait()

        plsc.subcore_barrier()

        fprev = fld(i - 1)
        for j in range(_BPF):
            drain = pltpu.make_async_copy(rows[j], out_at(fprev, j), wsem)
            if parity == 0:

                @pl.when(k >= 1)
                def _():
                    drain.wait()

            else:
                drain.wait()
            gather(sp, f, j, rows[j])

        nxt = sps[(parity + 1) % 2]
        if parity == 0:

            @pl.when(s == 0)
            def _():
                prefetch(i + 1, nxt)

        else:

            @pl.when(jnp.logical_and(s == 0, k < _NUM_FIELDS // 2 - 1))
            def _():
                prefetch(i + 1, nxt)

        for j in range(_BPF):
            pltpu.make_async_copy(sp.at[idx_at(f, j)], rows[j], gsem).wait()
            pltpu.async_copy(rows[j], out_at(f, j), wsem)

    def step(k, carry):
        field_body(2 * k, 0, k)
        field_body(2 * k + 1, 1, k)
        return carry

    lax.fori_loop(0, _NUM_FIELDS // 2, step, 0)

    lastf = fld(_NUM_FIELDS - 1)
    for j in range(_BPF):
        pltpu.make_async_copy(rows[j], out_at(lastf, j), wsem).wait()


@jax.jit
def kernel(x, weight):
    xt = x.T
    mesh = plsc.VectorSubcoreMesh(core_axis_name="c", subcore_axis_name="s")
    out = pl.kernel(
        _body,
        out_type=jax.ShapeDtypeStruct((_TOTAL, _EMBED_DIM), jnp.float32),
        mesh=mesh,
        scratch_types=[
            pltpu.VMEM((_NUM_FIELDS, _BATCH_W), jnp.int32),
            pltpu.VMEM_SHARED((_FIELD_DIM, _EMBED_DIM), jnp.float32),
            pltpu.VMEM_SHARED((_FIELD_DIM, _EMBED_DIM), jnp.float32),
            pltpu.VMEM((_BLK, _EMBED_DIM), jnp.float32),
            pltpu.VMEM((_BLK, _EMBED_DIM), jnp.float32),
            pltpu.VMEM((_BLK, _EMBED_DIM), jnp.float32),
            pltpu.VMEM((_BLK, _EMBED_DIM), jnp.float32),
            pltpu.SemaphoreType.DMA,
            pltpu.SemaphoreType.DMA,
            pltpu.SemaphoreType.DMA,
        ],
    )(xt, weight)
    return out.reshape(_NUM_FIELDS, _BATCH, _EMBED_DIM).transpose(1, 0, 2)


# speedup vs baseline: 1.0108x; 1.0108x over previous
import functools

import jax
import jax.numpy as jnp
import numpy as np
from jax import lax
from jax.experimental import pallas as pl
from jax.experimental.pallas import tpu as pltpu
from jax.experimental.pallas import tpu_sc as plsc

_EMBED_DIM = 128
_BATCH = 16384
_NUM_FIELDS = 26
_FIELD_DIM = 1000

_NC = 2
_NS = 16
_NW = _NC * _NS
_TOTAL = _BATCH * _NUM_FIELDS
_BATCH_W = _BATCH // _NW
_BLK = 64
_BPF = _BATCH_W // _BLK


def _body(x_hbm, w_hbm, out_hbm, xt_v, spA, spB, rows0, rows1, rows2, rows3,
          rows4, rows5, rows6, rows7, gsem, wsem, csem):
    c = lax.axis_index("c")
    s = lax.axis_index("s")
    wid = s * _NC + c
    b0 = wid * _BATCH_W
    rows = (rows0, rows1, rows2, rows3, rows4, rows5, rows6, rows7)
    sps = (spA, spB)

    pltpu.sync_copy(x_hbm.at[:, pl.ds(b0, _BATCH_W)], xt_v)

    def fld(i):
        return lax.rem(i + c * (_NUM_FIELDS // 2), _NUM_FIELDS)

    def idx_at(f, j):
        return xt_v.at[f, pl.ds(j * _BLK, _BLK)]

    def gather(sp, f, j, buf):
        pltpu.async_copy(sp.at[idx_at(f, j)], buf, gsem)

    def out_at(f, j):
        return out_hbm.at[pl.ds(f * _BATCH + b0 + j * _BLK, _BLK)]

    def prefetch(i, sp):
        pltpu.async_copy(
            w_hbm.at[pl.ds(fld(i) * _FIELD_DIM, _FIELD_DIM)], sp, csem)

    @pl.when(s == 0)
    def _():
        prefetch(0, sps[0])

    def field_body(i, parity, k):
        sp = sps[parity]
        f = fld(i)

        @pl.when(s == 0)
        def _():
            pltpu.make_async_copy(
                w_hbm.at[pl.ds(0, _FIELD_DIM)], sp, csem).wait()

        plsc.subcore_barrier()

        fprev = fld(i - 1)
        for j in range(_BPF):
            drain = pltpu.make_async_copy(rows[j], out_at(fprev, j), wsem)
            if parity == 0:

                @pl.when(k >= 1)
                def _():
                    drain.wait()

            else:
                drain.wait()
            gather(sp, f, j, rows[j])

        nxt = sps[(parity + 1) % 2]
        if parity == 0:

            @pl.when(s == 0)
            def _():
                prefetch(i + 1, nxt)

        else:

            @pl.when(jnp.logical_and(s == 0, k < _NUM_FIELDS // 2 - 1))
            def _():
                prefetch(i + 1, nxt)

        for j in range(_BPF):
            pltpu.make_async_copy(sp.at[idx_at(f, j)], rows[j], gsem).wait()
            pltpu.async_copy(rows[j], out_at(f, j), wsem)

    def step(k, carry):
        field_body(2 * k, 0, k)
        field_body(2 * k + 1, 1, k)
        return carry

    lax.fori_loop(0, _NUM_FIELDS // 2, step, 0)

    lastf = fld(_NUM_FIELDS - 1)
    for j in range(_BPF):
        pltpu.make_async_copy(rows[j], out_at(lastf, j), wsem).wait()


@jax.jit
def kernel(x, weight):
    xt = x.T
    mesh = plsc.VectorSubcoreMesh(core_axis_name="c", subcore_axis_name="s")
    out = pl.kernel(
        _body,
        out_type=jax.ShapeDtypeStruct((_TOTAL, _EMBED_DIM), jnp.float32),
        mesh=mesh,
        scratch_types=[
            pltpu.VMEM((_NUM_FIELDS, _BATCH_W), jnp.int32),
            pltpu.VMEM_SHARED((_FIELD_DIM, _EMBED_DIM), jnp.float32),
            pltpu.VMEM_SHARED((_FIELD_DIM, _EMBED_DIM), jnp.float32),
            pltpu.VMEM((_BLK, _EMBED_DIM), jnp.float32),
            pltpu.VMEM((_BLK, _EMBED_DIM), jnp.float32),
            pltpu.VMEM((_BLK, _EMBED_DIM), jnp.float32),
            pltpu.VMEM((_BLK, _EMBED_DIM), jnp.float32),
            pltpu.VMEM((_BLK, _EMBED_DIM), jnp.float32),
            pltpu.VMEM((_BLK, _EMBED_DIM), jnp.float32),
            pltpu.VMEM((_BLK, _EMBED_DIM), jnp.float32),
            pltpu.VMEM((_BLK, _EMBED_DIM), jnp.float32),
            pltpu.SemaphoreType.DMA,
            pltpu.SemaphoreType.DMA,
            pltpu.SemaphoreType.DMA,
        ],
    )(xt, weight)
    return out.reshape(_NUM_FIELDS, _BATCH, _EMBED_DIM).transpose(1, 0, 2)
